# Initial kernel scaffold; baseline (speedup 1.0000x reference)
import jax
import jax.numpy as jnp
from jax.experimental import pallas as pl


def kernel(coordinates, width, height, top_left_x, top_left_y, bottom_right_x, bottom_right_y, width_emb, height_emb):
    raise NotImplementedError("write your pallas kernel here")


# speedup vs baseline: 1.4896x; 1.4896x over previous
import functools

import jax
import jax.numpy as jnp
from jax import lax
from jax.experimental import pallas as pl
from jax.experimental.pallas import tpu as pltpu
from jax.experimental.pallas import tpu_sc as plsc

B, L, D = 4, 4096, 768
MAX_POS = 1024
NC, NS, LANES = 2, 16, 16
NW = NC * NS
TOK = B * L
TPW = TOK // NW
C = 32
NCH = TPW // C
DV = D // LANES


def _sc_body(c0, c1, c2, c3, w, h, tlx, tly, brx, bry, wemb, hemb, out,
             idx_v, stage, whrow, wv, hv, isem, gsem, ssem):
    wid = lax.axis_index("s") * NC + lax.axis_index("c")
    base = pl.multiple_of(wid * TPW, TPW)
    b = wid // (NW // B)

    descs = [pltpu.async_copy(c.at[pl.ds(base, TPW)], idx_v.at[t], isem)
             for t, c in enumerate((c0, c1, c2, c3))]
    for d in descs:
        d.wait()
    for t in range(4):
        for j in range(TPW // LANES):
            sl = pl.ds(j * LANES, LANES)
            idx_v[t, sl] = jnp.minimum(idx_v[t, sl], MAX_POS - 1)

    bvec = jnp.full((LANES,), b, jnp.int32)
    pltpu.async_copy(w.at[bvec], wv, gsem).wait()
    pltpu.async_copy(h.at[bvec], hv, gsem).wait()
    wb = jnp.minimum(wv[...], MAX_POS - 1)
    hb = jnp.minimum(hv[...], MAX_POS - 1)
    pltpu.async_copy(wemb.at[wb], stage.at[0, pl.ds(0, LANES)], gsem).wait()
    pltpu.async_copy(hemb.at[hb], stage.at[1, pl.ds(0, LANES)], gsem).wait()
    for s in range(DV):
        sl = pl.ds(s * LANES, LANES)
        whrow[sl] = stage[0, 0, sl] + stage[1, 0, sl]

    def chunk(k, carry):
        koff = pl.multiple_of(k * C, C)
        ksl = pl.ds(koff, C)
        descs = [pltpu.async_copy(tab.at[idx_v.at[t, ksl]], stage.at[t], gsem)
                 for t, tab in enumerate((tlx, tly, brx, bry))]
        for d in descs:
            d.wait()

        def combine(i, carry2):
            for s in range(DV):
                sl = pl.ds(s * LANES, LANES)
                stage[0, i, sl] = (stage[0, i, sl] + stage[1, i, sl]
                                   + stage[2, i, sl] + stage[3, i, sl]
                                   + whrow[sl])
            return carry2
        lax.fori_loop(0, C, combine, 0)

        pltpu.async_copy(stage.at[0],
                         out.at[pl.ds(base + koff, C)], ssem).wait()
        return carry
    lax.fori_loop(0, NCH, chunk, 0)


_mesh = plsc.VectorSubcoreMesh(core_axis_name="c", subcore_axis_name="s")

_spatial_sum = functools.partial(
    pl.kernel,
    out_type=jax.ShapeDtypeStruct((TOK, D), jnp.float32),
    mesh=_mesh,
    scratch_types=[
        pltpu.VMEM((4, TPW), jnp.int32),
        pltpu.VMEM((4, C, D), jnp.float32),
        pltpu.VMEM((D,), jnp.float32),
        pltpu.VMEM((LANES,), jnp.int32),
        pltpu.VMEM((LANES,), jnp.int32),
        pltpu.SemaphoreType.DMA,
        pltpu.SemaphoreType.DMA,
        pltpu.SemaphoreType.DMA,
    ],
)(_sc_body)


def kernel(coordinates, width, height, top_left_x, top_left_y,
           bottom_right_x, bottom_right_y, width_emb, height_emb):
    cidx = coordinates.reshape(TOK, 4).astype(jnp.int32)
    c0, c1, c2, c3 = (cidx[:, t] for t in range(4))
    out = _spatial_sum(c0, c1, c2, c3,
                       width.astype(jnp.int32), height.astype(jnp.int32),
                       top_left_x, top_left_y, bottom_right_x, bottom_right_y,
                       width_emb, height_emb)
    return out.reshape(B, L, D)


# speedup vs baseline: 1.7317x; 1.1625x over previous
import functools

import jax
import jax.numpy as jnp
from jax import lax
from jax.experimental import pallas as pl
from jax.experimental.pallas import tpu as pltpu
from jax.experimental.pallas import tpu_sc as plsc

B, L, D = 4, 4096, 768
MAX_POS = 1024
NC, NS, LANES = 2, 16, 16
NW = NC * NS
TOK = B * L
TPW = TOK // NW
C = 16
NCH = TPW // C
DV = D // LANES


def _sc_body(c0, c1, c2, c3, w, h, tlx, tly, brx, bry, wemb, hemb, out,
             idx_v, stage, whrow, wv, hv, isem, gsem0, gsem1, ssem0, ssem1):
    wid = lax.axis_index("s") * NC + lax.axis_index("c")
    base = pl.multiple_of(wid * TPW, TPW)
    b = wid // (NW // B)

    descs = [pltpu.async_copy(c.at[pl.ds(base, TPW)], idx_v.at[t], isem)
             for t, c in enumerate((c0, c1, c2, c3))]
    for d in descs:
        d.wait()
    for t in range(4):
        for j in range(TPW // LANES):
            sl = pl.ds(j * LANES, LANES)
            idx_v[t, sl] = jnp.minimum(idx_v[t, sl], MAX_POS - 1)

    bvec = jnp.full((LANES,), b, jnp.int32)
    pltpu.async_copy(w.at[bvec], wv, isem).wait()
    pltpu.async_copy(h.at[bvec], hv, isem).wait()
    wb = jnp.minimum(wv[...], MAX_POS - 1)
    hb = jnp.minimum(hv[...], MAX_POS - 1)
    pltpu.async_copy(wemb.at[wb], stage.at[0, 0, pl.ds(0, LANES)], isem).wait()
    pltpu.async_copy(hemb.at[hb], stage.at[0, 1, pl.ds(0, LANES)], isem).wait()
    for s in range(DV):
        sl = pl.ds(s * LANES, LANES)
        whrow[sl] = stage[0, 0, 0, sl] + stage[0, 1, 0, sl]

    gsems = (gsem0, gsem1)
    ssems = (ssem0, ssem1)

    def gather_descs(k, p, sem):
        ksl = pl.ds(pl.multiple_of(k * C, C), C)
        return [pltpu.make_async_copy(tab.at[idx_v.at[t, ksl]],
                                      stage.at[p, t], sem)
                for t, tab in enumerate((tlx, tly, brx, bry))]

    def issue_gathers(k, p, sem):
        for d in gather_descs(k, p, sem):
            d.start()

    def wait_gathers(k, p, sem):
        for d in gather_descs(k, p, sem):
            d.wait()

    def scatter_desc(k, p, sem):
        koff = pl.multiple_of(k * C, C)
        return pltpu.make_async_copy(stage.at[p, 0],
                                     out.at[pl.ds(base + koff, C)], sem)

    def combine(p):
        def body(i, carry):
            for s in range(DV):
                sl = pl.ds(s * LANES, LANES)
                stage[p, 0, i, sl] = (stage[p, 0, i, sl] + stage[p, 1, i, sl]
                                      + stage[p, 2, i, sl] + stage[p, 3, i, sl]
                                      + whrow[sl])
            return carry
        lax.fori_loop(0, C, body, 0)

    issue_gathers(0, 0, gsem0)

    def pair(j, carry):
        k0 = pl.multiple_of(2 * j, 2)
        k1 = k0 + 1
        wait_gathers(k0, 0, gsem0)

        @pl.when(j > 0)
        def _():
            scatter_desc(k0, 1, ssem1).wait()
        issue_gathers(k1, 1, gsem1)
        combine(0)
        scatter_desc(k0, 0, ssem0).start()
        wait_gathers(k1, 1, gsem1)
        scatter_desc(k0, 0, ssem0).wait()

        @pl.when(k1 + 1 < NCH)
        def _():
            issue_gathers(k1 + 1, 0, gsem0)
        combine(1)
        scatter_desc(k1, 1, ssem1).start()
        return carry
    lax.fori_loop(0, NCH // 2, pair, 0)
    scatter_desc(NCH - 1, 1, ssem1).wait()


_mesh = plsc.VectorSubcoreMesh(core_axis_name="c", subcore_axis_name="s")

_spatial_sum = functools.partial(
    pl.kernel,
    out_type=jax.ShapeDtypeStruct((TOK, D), jnp.float32),
    mesh=_mesh,
    scratch_types=[
        pltpu.VMEM((4, TPW), jnp.int32),
        pltpu.VMEM((2, 4, C, D), jnp.float32),
        pltpu.VMEM((D,), jnp.float32),
        pltpu.VMEM((LANES,), jnp.int32),
        pltpu.VMEM((LANES,), jnp.int32),
        pltpu.SemaphoreType.DMA,
        pltpu.SemaphoreType.DMA,
        pltpu.SemaphoreType.DMA,
        pltpu.SemaphoreType.DMA,
        pltpu.SemaphoreType.DMA,
    ],
)(_sc_body)


def kernel(coordinates, width, height, top_left_x, top_left_y,
           bottom_right_x, bottom_right_y, width_emb, height_emb):
    cidx = coordinates.reshape(TOK, 4).astype(jnp.int32)
    c0, c1, c2, c3 = (cidx[:, t] for t in range(4))
    out = _spatial_sum(c0, c1, c2, c3,
                       width.astype(jnp.int32), height.astype(jnp.int32),
                       top_left_x, top_left_y, bottom_right_x, bottom_right_y,
                       width_emb, height_emb)
    return out.reshape(B, L, D)
